# Initial kernel scaffold; baseline (speedup 1.0000x reference)
import functools

import jax
import jax.numpy as jnp
from jax import lax
from jax.experimental import pallas as pl
from jax.experimental.pallas import tpu as pltpu
from jax.experimental.pallas import tpu_sc as plsc

N = 10000
E = 320000
D = 128
H = 256
NC, NS = 2, 16
NW = NC * NS
CHUNK = 128
NCHUNK = 80
EPW = NCHUNK * CHUNK
EPAD = EPW * NW
NPAD = 10240
STRIPE = NPAD // NS
DUMP = N

_mesh = plsc.VectorSubcoreMesh(core_axis_name="c", subcore_axis_name="s")


@functools.partial(
    pl.kernel,
    mesh=_mesh,
    out_type=jax.ShapeDtypeStruct((NC, 2, NPAD, 16), jnp.float32),
    scratch_types=[
        pltpu.VMEM((NCHUNK, CHUNK), jnp.int32),
        pltpu.VMEM((NCHUNK, CHUNK), jnp.int32),
        pltpu.VMEM((CHUNK, 16), jnp.float32),
        pltpu.VMEM_SHARED((2, NPAD, 16), jnp.float32),
    ],
)
def _deg_kernel(src_hbm, dst_hbm, ones_hbm, zeros_hbm, out_hbm,
                src_v, dst_v, ones_v, deg_sh):
    c = lax.axis_index("c")
    s = lax.axis_index("s")
    wid = c * NS + s
    pltpu.sync_copy(src_hbm.at[wid], src_v)
    pltpu.sync_copy(dst_hbm.at[wid], dst_v)
    pltpu.sync_copy(ones_hbm, ones_v)
    pltpu.sync_copy(zeros_hbm, deg_sh.at[0, pl.ds(s * STRIPE, STRIPE)])
    pltpu.sync_copy(zeros_hbm, deg_sh.at[1, pl.ds(s * STRIPE, STRIPE)])
    plsc.subcore_barrier()

    def body(j, carry):
        pltpu.sync_copy(ones_v, deg_sh.at[0, src_v.at[j]], add=True)
        pltpu.sync_copy(ones_v, deg_sh.at[1, dst_v.at[j]], add=True)
        return carry

    lax.fori_loop(0, NCHUNK, body, 0)
    plsc.subcore_barrier()
    pltpu.sync_copy(deg_sh.at[0, pl.ds(s * STRIPE, STRIPE)],
                    out_hbm.at[c, 0, pl.ds(s * STRIPE, STRIPE)])
    pltpu.sync_copy(deg_sh.at[1, pl.ds(s * STRIPE, STRIPE)],
                    out_hbm.at[c, 1, pl.ds(s * STRIPE, STRIPE)])


@functools.partial(
    pl.kernel,
    mesh=_mesh,
    out_type=jax.ShapeDtypeStruct((NC, NPAD, D), jnp.float32),
    scratch_types=[
        pltpu.VMEM((NCHUNK, CHUNK), jnp.int32),
        pltpu.VMEM((NCHUNK, CHUNK), jnp.int32),
        pltpu.VMEM((CHUNK, D), jnp.float32),
        pltpu.VMEM((CHUNK, D), jnp.float32),
        pltpu.VMEM_SHARED((NPAD, D), jnp.float32),
        pltpu.SemaphoreType.DMA,
        pltpu.SemaphoreType.DMA,
    ],
)
def _agg_kernel(src_hbm, dst_hbm, table_hbm, zeros_hbm, out_hbm,
                src_v, dst_v, rows0_v, rows1_v, acc_sh, sem0, sem1):
    c = lax.axis_index("c")
    s = lax.axis_index("s")
    wid = c * NS + s
    pltpu.sync_copy(src_hbm.at[wid], src_v)
    pltpu.sync_copy(dst_hbm.at[wid], dst_v)
    pltpu.sync_copy(zeros_hbm, acc_sh.at[pl.ds(s * STRIPE, STRIPE)])
    plsc.subcore_barrier()

    pltpu.async_copy(table_hbm.at[src_v.at[0]], rows0_v, sem0)

    def body(j2, carry):
        j = j2 * 2
        pltpu.async_copy(table_hbm.at[src_v.at[j + 1]], rows1_v, sem1)
        pltpu.make_async_copy(table_hbm.at[src_v.at[j]], rows0_v, sem0).wait()
        pltpu.sync_copy(rows0_v, acc_sh.at[dst_v.at[j]], add=True)

        @pl.when(j2 < NCHUNK // 2 - 1)
        def _():
            pltpu.async_copy(table_hbm.at[src_v.at[j + 2]], rows0_v, sem0)

        pltpu.make_async_copy(table_hbm.at[src_v.at[j + 1]], rows1_v, sem1).wait()
        pltpu.sync_copy(rows1_v, acc_sh.at[dst_v.at[j + 1]], add=True)
        return carry

    lax.fori_loop(0, NCHUNK // 2, body, 0)
    plsc.subcore_barrier()
    pltpu.sync_copy(acc_sh.at[pl.ds(s * STRIPE, STRIPE)],
                    out_hbm.at[c, pl.ds(s * STRIPE, STRIPE)])


_TCR = 256
_TCG = NPAD // _TCR


def _norm_from(deg_ref, table):
    d = deg_ref[0, table] + deg_ref[1, table]
    return lax.rsqrt(jnp.clip(d[:, 0:1], 1.0, None))


def _scale_body(x_ref, deg_ref, out_ref):
    out_ref[...] = x_ref[...] * _norm_from(deg_ref, 0)


def _mid_body(agg_ref, deg_ref, w1_ref, b1_ref, w2_ref, out_ref):
    a = (agg_ref[0] + agg_ref[1]) * _norm_from(deg_ref, 1)
    h = jnp.dot(a, w1_ref[...], preferred_element_type=jnp.float32) + b1_ref[...]
    r = jnp.maximum(h, 0.0) * _norm_from(deg_ref, 0)
    out_ref[...] = jnp.dot(r, w2_ref[...], preferred_element_type=jnp.float32)


def _final_body(agg_ref, deg_ref, b2_ref, out_ref):
    a = (agg_ref[0] + agg_ref[1]) * _norm_from(deg_ref, 1)
    out_ref[...] = a + b2_ref[...]


def _deg_spec():
    return pl.BlockSpec((NC, 2, _TCR, 16), lambda i: (0, 0, i, 0))


def _tc_scale(x_pad, degs):
    return pl.pallas_call(
        _scale_body,
        grid=(_TCG,),
        in_specs=[pl.BlockSpec((_TCR, D), lambda i: (i, 0)), _deg_spec()],
        out_specs=pl.BlockSpec((_TCR, D), lambda i: (i, 0)),
        out_shape=jax.ShapeDtypeStruct((NPAD, D), jnp.float32),
    )(x_pad, degs)


def _tc_mid(agg, degs, W1, b1, W2):
    return pl.pallas_call(
        _mid_body,
        grid=(_TCG,),
        in_specs=[
            pl.BlockSpec((NC, _TCR, D), lambda i: (0, i, 0)),
            _deg_spec(),
            pl.BlockSpec((D, H), lambda i: (0, 0)),
            pl.BlockSpec((1, H), lambda i: (0, 0)),
            pl.BlockSpec((H, D), lambda i: (0, 0)),
        ],
        out_specs=pl.BlockSpec((_TCR, D), lambda i: (i, 0)),
        out_shape=jax.ShapeDtypeStruct((NPAD, D), jnp.float32),
    )(agg, degs, W1, b1, W2)


def _tc_final(agg, degs, b2):
    return pl.pallas_call(
        _final_body,
        grid=(_TCG,),
        in_specs=[
            pl.BlockSpec((NC, _TCR, D), lambda i: (0, i, 0)),
            _deg_spec(),
            pl.BlockSpec((1, D), lambda i: (0, 0)),
        ],
        out_specs=pl.BlockSpec((_TCR, D), lambda i: (i, 0)),
        out_shape=jax.ShapeDtypeStruct((NPAD, D), jnp.float32),
    )(agg, degs, b2)


def kernel(x, edge_index, W1, b1, W2, b2):
    src = edge_index[0].astype(jnp.int32)
    dst = edge_index[1].astype(jnp.int32)
    pad = jnp.full((EPAD - E,), DUMP, jnp.int32)
    srcp = jnp.concatenate([src, pad]).reshape(NW, NCHUNK, CHUNK)
    dstp = jnp.concatenate([dst, pad]).reshape(NW, NCHUNK, CHUNK)
    x_pad = jnp.zeros((NPAD, D), jnp.float32).at[:N].set(x)
    zeros_rows = jnp.zeros((STRIPE, D), jnp.float32)
    zeros16 = jnp.zeros((STRIPE, 16), jnp.float32)
    ones16 = jnp.ones((CHUNK, 16), jnp.float32)

    degs = _deg_kernel(srcp, dstp, ones16, zeros16)
    xs = _tc_scale(x_pad, degs)
    agg1 = _agg_kernel(srcp, dstp, xs, zeros_rows)
    p = _tc_mid(agg1, degs, W1, b1.reshape(1, H), W2)
    agg2 = _agg_kernel(srcp, dstp, p, zeros_rows)
    out = _tc_final(agg2, degs, b2.reshape(1, D))
    return out[:N]


# speedup vs baseline: 4.2049x; 4.2049x over previous
import functools

import jax
import jax.numpy as jnp
from jax import lax
from jax.experimental import pallas as pl
from jax.experimental.pallas import tpu as pltpu
from jax.experimental.pallas import tpu_sc as plsc

N = 10000
E = 320000
D = 128
DH = 64
H = 256
NC, NS = 2, 16
NW = NC * NS
CHUNK = 128
NCHUNK = 80
EPW = NCHUNK * CHUNK
EPAD = EPW * NW
NPAD = 10240
STRIPE = NPAD // NS
DUMP = N

_mesh = plsc.VectorSubcoreMesh(core_axis_name="c", subcore_axis_name="s")


@functools.partial(
    pl.kernel,
    mesh=_mesh,
    compiler_params=pltpu.CompilerParams(use_tc_tiling_on_sc=False),
    out_type=jax.ShapeDtypeStruct((NC, 2, NPAD, 16), jnp.float32),
    scratch_types=[
        pltpu.VMEM((NCHUNK, CHUNK), jnp.int32),
        pltpu.VMEM((NCHUNK, CHUNK), jnp.int32),
        pltpu.VMEM((CHUNK, 16), jnp.float32),
        pltpu.VMEM_SHARED((NPAD, 16), jnp.float32),
        pltpu.VMEM_SHARED((NPAD, 16), jnp.float32),
        pltpu.SemaphoreType.DMA,
    ],
)
def _deg_kernel(src_hbm, dst_hbm, ones_hbm, zeros_hbm, out_hbm,
                src_v, dst_v, ones_v, dsrc_sh, ddst_sh, sem):
    c = lax.axis_index("c")
    s = lax.axis_index("s")
    wid = c * NS + s
    pltpu.sync_copy(src_hbm.at[wid], src_v)
    pltpu.sync_copy(dst_hbm.at[wid], dst_v)
    pltpu.sync_copy(ones_hbm, ones_v)
    pltpu.sync_copy(zeros_hbm, dsrc_sh.at[pl.ds(s * STRIPE, STRIPE)])
    pltpu.sync_copy(zeros_hbm, ddst_sh.at[pl.ds(s * STRIPE, STRIPE)])
    plsc.subcore_barrier()

    def fire(j, carry):
        pltpu.async_copy(ones_v, dsrc_sh.at[src_v.at[j]], sem, add=True)
        pltpu.async_copy(ones_v, ddst_sh.at[dst_v.at[j]], sem, add=True)
        return carry

    def drain(j, carry):
        pltpu.make_async_copy(ones_v, dsrc_sh.at[src_v.at[0]], sem).wait()
        pltpu.make_async_copy(ones_v, ddst_sh.at[dst_v.at[0]], sem).wait()
        return carry

    lax.fori_loop(0, NCHUNK, fire, 0)
    lax.fori_loop(0, NCHUNK, drain, 0)
    plsc.subcore_barrier()
    pltpu.sync_copy(dsrc_sh.at[pl.ds(s * STRIPE, STRIPE)],
                    out_hbm.at[c, 0, pl.ds(s * STRIPE, STRIPE)])
    pltpu.sync_copy(ddst_sh.at[pl.ds(s * STRIPE, STRIPE)],
                    out_hbm.at[c, 1, pl.ds(s * STRIPE, STRIPE)])


@functools.partial(
    pl.kernel,
    mesh=_mesh,
    compiler_params=pltpu.CompilerParams(use_tc_tiling_on_sc=False),
    out_type=jax.ShapeDtypeStruct((2, NC, NPAD, DH), jnp.float32),
    scratch_types=[
        pltpu.VMEM((NCHUNK, CHUNK), jnp.int32),
        pltpu.VMEM((NCHUNK, CHUNK), jnp.int32),
        pltpu.VMEM((CHUNK, DH), jnp.float32),
        pltpu.VMEM((CHUNK, DH), jnp.float32),
        pltpu.VMEM_SHARED((NPAD, DH), jnp.float32),
        pltpu.SemaphoreType.DMA,
        pltpu.SemaphoreType.DMA,
    ],
)
def _agg_kernel(src_hbm, dst_hbm, table0_hbm, table1_hbm, zeros_hbm, out_hbm,
                src_v, dst_v, rows0_v, rows1_v, acc_sh, sem0, sem1):
    c = lax.axis_index("c")
    s = lax.axis_index("s")
    wid = c * NS + s
    pltpu.sync_copy(src_hbm.at[wid], src_v)
    pltpu.sync_copy(dst_hbm.at[wid], dst_v)

    for h, table_hbm in ((0, table0_hbm), (1, table1_hbm)):
        pltpu.sync_copy(zeros_hbm, acc_sh.at[pl.ds(s * STRIPE, STRIPE)])
        plsc.subcore_barrier()

        pltpu.async_copy(table_hbm.at[src_v.at[0]], rows0_v, sem0)

        def body(j2, carry):
            j = j2 * 2
            pltpu.async_copy(table_hbm.at[src_v.at[j + 1]], rows1_v, sem1)
            pltpu.make_async_copy(table_hbm.at[src_v.at[j]], rows0_v,
                                  sem0).wait()
            pltpu.sync_copy(rows0_v, acc_sh.at[dst_v.at[j]], add=True)

            @pl.when(j2 < NCHUNK // 2 - 1)
            def _():
                pltpu.async_copy(table_hbm.at[src_v.at[j + 2]], rows0_v, sem0)

            pltpu.make_async_copy(table_hbm.at[src_v.at[j + 1]], rows1_v,
                                  sem1).wait()
            pltpu.sync_copy(rows1_v, acc_sh.at[dst_v.at[j + 1]], add=True)
            return carry

        lax.fori_loop(0, NCHUNK // 2, body, 0)
        plsc.subcore_barrier()
        pltpu.sync_copy(acc_sh.at[pl.ds(s * STRIPE, STRIPE)],
                        out_hbm.at[h, c, pl.ds(s * STRIPE, STRIPE)])
        plsc.subcore_barrier()


_TCR = 256
_TCG = NPAD // _TCR


def _norm_from(deg_ref, table):
    d = deg_ref[0, table] + deg_ref[1, table]
    return lax.rsqrt(jnp.clip(d[:, 0:1], 1.0, None))


def _merge_agg(agg_ref):
    return jnp.concatenate(
        [agg_ref[0, 0] + agg_ref[0, 1], agg_ref[1, 0] + agg_ref[1, 1]],
        axis=1)


def _scale_body(x_ref, deg_ref, out0_ref, out1_ref):
    xs = x_ref[...] * _norm_from(deg_ref, 0)
    out0_ref[...] = xs[:, :DH]
    out1_ref[...] = xs[:, DH:]


def _mid_body(agg_ref, deg_ref, w1_ref, b1_ref, w2_ref, out0_ref, out1_ref):
    a = _merge_agg(agg_ref) * _norm_from(deg_ref, 1)
    h = jnp.dot(a, w1_ref[...], preferred_element_type=jnp.float32) + b1_ref[...]
    r = jnp.maximum(h, 0.0) * _norm_from(deg_ref, 0)
    p = jnp.dot(r, w2_ref[...], preferred_element_type=jnp.float32)
    out0_ref[...] = p[:, :DH]
    out1_ref[...] = p[:, DH:]


def _final_body(agg_ref, deg_ref, b2_ref, out_ref):
    a = _merge_agg(agg_ref) * _norm_from(deg_ref, 1)
    out_ref[...] = a + b2_ref[...]


def _deg_spec():
    return pl.BlockSpec((NC, 2, _TCR, 16), lambda i: (0, 0, i, 0))


def _agg_spec():
    return pl.BlockSpec((2, NC, _TCR, DH), lambda i: (0, 0, i, 0))


def _half_specs():
    return [pl.BlockSpec((_TCR, DH), lambda i: (i, 0)) for _ in range(2)]


def _half_shapes():
    return [jax.ShapeDtypeStruct((NPAD, DH), jnp.float32) for _ in range(2)]


def _tc_scale(x_pad, degs):
    return pl.pallas_call(
        _scale_body,
        grid=(_TCG,),
        in_specs=[pl.BlockSpec((_TCR, D), lambda i: (i, 0)), _deg_spec()],
        out_specs=_half_specs(),
        out_shape=_half_shapes(),
    )(x_pad, degs)


def _tc_mid(agg, degs, W1, b1, W2):
    return pl.pallas_call(
        _mid_body,
        grid=(_TCG,),
        in_specs=[
            _agg_spec(),
            _deg_spec(),
            pl.BlockSpec((D, H), lambda i: (0, 0)),
            pl.BlockSpec((1, H), lambda i: (0, 0)),
            pl.BlockSpec((H, D), lambda i: (0, 0)),
        ],
        out_specs=_half_specs(),
        out_shape=_half_shapes(),
    )(agg, degs, W1, b1, W2)


def _tc_final(agg, degs, b2):
    return pl.pallas_call(
        _final_body,
        grid=(_TCG,),
        in_specs=[
            _agg_spec(),
            _deg_spec(),
            pl.BlockSpec((1, D), lambda i: (0, 0)),
        ],
        out_specs=pl.BlockSpec((_TCR, D), lambda i: (i, 0)),
        out_shape=jax.ShapeDtypeStruct((NPAD, D), jnp.float32),
    )(agg, degs, b2)


def kernel(x, edge_index, W1, b1, W2, b2):
    src = edge_index[0].astype(jnp.int32)
    dst = edge_index[1].astype(jnp.int32)
    pad = jnp.full((EPAD - E,), DUMP, jnp.int32)
    srcp = jnp.concatenate([src, pad]).reshape(NW, NCHUNK, CHUNK)
    dstp = jnp.concatenate([dst, pad]).reshape(NW, NCHUNK, CHUNK)
    x_pad = jnp.zeros((NPAD, D), jnp.float32).at[:N].set(x)
    zeros_rows = jnp.zeros((STRIPE, DH), jnp.float32)
    zeros16 = jnp.zeros((STRIPE, 16), jnp.float32)
    ones16 = jnp.ones((CHUNK, 16), jnp.float32)

    degs = _deg_kernel(srcp, dstp, ones16, zeros16)
    xs0, xs1 = _tc_scale(x_pad, degs)
    agg1 = _agg_kernel(srcp, dstp, xs0, xs1, zeros_rows)
    p0, p1 = _tc_mid(agg1, degs, W1, b1.reshape(1, H), W2)
    agg2 = _agg_kernel(srcp, dstp, p0, p1, zeros_rows)
    out = _tc_final(agg2, degs, b2.reshape(1, D))
    return out[:N]
